# baseline (speedup 1.0000x reference)
import functools

import jax
import jax.numpy as jnp
from jax.experimental import pallas as pl


def _layer_body(a_ref, emb_ref, out_ref):
    i = pl.program_id(0)
    bm = out_ref.shape[0]
    a = a_ref[...]
    acc = jnp.dot(a, emb_ref[...],
                  preferred_element_type=jnp.float32)
    rowsum = jnp.sum(a, axis=1, keepdims=True)
    emb_rows = emb_ref[pl.ds(i * bm, bm), :]
    out_ref[...] = (emb_rows + acc) * (1.0 / (rowsum + 1.0))


@functools.partial(jax.jit, static_argnames=("bm",))
def _layer(A, emb, bm):
    N, D = emb.shape
    return pl.pallas_call(
        _layer_body,
        grid=(N // bm,),
        in_specs=[
            pl.BlockSpec((bm, N), lambda i: (i, 0)),
            pl.BlockSpec((N, D), lambda i: (0, 0),
                         pipeline_mode=pl.Buffered(buffer_count=1)),
        ],
        out_specs=pl.BlockSpec((bm, D), lambda i: (i, 0)),
        out_shape=jax.ShapeDtypeStruct((N, D), jnp.float32),
    )(A, emb)


def kernel(embedding, bottom_to_top_paths):
    emb = embedding
    P = bottom_to_top_paths.shape[0]
    N = emb.shape[0]
    bm = 200 if N % 200 == 0 else 8
    for i in range(P):
        emb = _layer(bottom_to_top_paths[i], emb, bm)
    return emb


# speedup vs baseline: 1.0193x; 1.0193x over previous
import functools

import jax
import jax.numpy as jnp
from jax.experimental import pallas as pl
import jax.experimental.pallas.tpu as pltpu


def _layer_body(a_ref, emb_ref, out_ref):
    i = pl.program_id(0)
    bm = out_ref.shape[0]
    a = a_ref[...]
    acc = jnp.dot(a, emb_ref[...],
                  preferred_element_type=jnp.float32)
    rowsum = jnp.sum(a, axis=1, keepdims=True)
    emb_rows = emb_ref[pl.ds(i * bm, bm), :]
    out_ref[...] = (emb_rows + acc) * (1.0 / (rowsum + 1.0))


@functools.partial(jax.jit, static_argnames=("bm",))
def _layer(A, emb, bm):
    N, D = emb.shape
    return pl.pallas_call(
        _layer_body,
        grid=(N // bm,),
        in_specs=[
            pl.BlockSpec((bm, N), lambda i: (i, 0)),
            pl.BlockSpec((N, D), lambda i: (0, 0),
                         pipeline_mode=pl.Buffered(buffer_count=1)),
        ],
        out_specs=pl.BlockSpec((bm, D), lambda i: (i, 0)),
        out_shape=jax.ShapeDtypeStruct((N, D), jnp.float32),
        compiler_params=pltpu.CompilerParams(
            dimension_semantics=("arbitrary",),
            vmem_limit_bytes=64 * 1024 * 1024,
        ),
    )(A, emb)


def kernel(embedding, bottom_to_top_paths):
    emb = embedding
    P = bottom_to_top_paths.shape[0]
    N = emb.shape[0]
    bm = 400 if N % 400 == 0 else 8
    for i in range(P):
        emb = _layer(bottom_to_top_paths[i], emb, bm)
    return emb
